# baseline (speedup 1.0000x reference)
import functools

import jax
import jax.numpy as jnp
from jax import lax
from jax.experimental import pallas as pl
from jax.experimental.pallas import tpu as pltpu
from jax.experimental.pallas import tpu_sc as plsc

N_NODES = 10000
N_EDGES = 320000
N_GRAPHS = 64
N_FEAT_OUT = 36

R = 10112
NC, NS = 2, 16
NW = NC * NS
CHUNK = 128
EROWS = 2560
ROWS_PW = EROWS // NW
RPS = R // NS

_SELU_ALPHA = 1.6732632423543772
_SELU_SCALE = 1.0507009873554805


def _selu(v):
    return _SELU_SCALE * jnp.where(v > 0, v, _SELU_ALPHA * (jnp.exp(v) - 1.0))


GRP = 8


def _sc_scatter_body(table, src2d, dst2d, zeros, out, acc, sidx, didx, rows,
                     gsem, ssem):
    c = lax.axis_index("c")
    s = lax.axis_index("s")
    wid = s * NC + c

    pltpu.sync_copy(zeros.at[pl.ds(s * RPS, RPS)], acc.at[pl.ds(s * RPS, RPS)])

    base = wid * ROWS_PW
    pltpu.sync_copy(src2d.at[pl.ds(base, ROWS_PW)], sidx)
    pltpu.sync_copy(dst2d.at[pl.ds(base, ROWS_PW)], didx)
    plsc.subcore_barrier()

    def _buf(half, j):
        return rows.at[pl.ds((half * GRP + j) * CHUNK, CHUNK)]

    def _fire_gathers(k0, half):
        return [
            pltpu.async_copy(table.at[sidx.at[k0 + j]], _buf(half, j), gsem)
            for j in range(GRP)
        ]

    def _scatter_half(k0, half, gd):
        for j in range(GRP):
            gd[j].wait()
            pltpu.async_copy(_buf(half, j), acc.at[didx.at[k0 + j]], ssem,
                             add=True)

    def _drain_scatters():
        for j in range(GRP):
            pltpu.make_async_copy(table.at[pl.ds(0, CHUNK)], _buf(0, j),
                                  ssem).wait()

    @pl.loop(0, ROWS_PW, step=2 * GRP)
    def _group(k0):
        @pl.when(k0 > 0)
        def _():
            _drain_scatters()
        gda = _fire_gathers(k0, 0)
        _scatter_half(k0, 0, gda)
        gdb = _fire_gathers(k0 + GRP, 1)
        _drain_scatters()
        _scatter_half(k0 + GRP, 1, gdb)

    _drain_scatters()

    plsc.subcore_barrier()
    pltpu.sync_copy(acc.at[pl.ds(s * RPS, RPS)], out.at[c, pl.ds(s * RPS, RPS)])


def _sc_scatter(dpad, table, src2d, dst2d, zeros):
    mesh = plsc.VectorSubcoreMesh(
        core_axis_name="c", subcore_axis_name="s", num_cores=NC, num_subcores=NS)
    f = pl.kernel(
        _sc_scatter_body,
        out_type=jax.ShapeDtypeStruct((NC, R, dpad), jnp.float32),
        mesh=mesh,
        scratch_types=[
            pltpu.VMEM_SHARED((R, dpad), jnp.float32),
            pltpu.VMEM((ROWS_PW, CHUNK), jnp.int32),
            pltpu.VMEM((ROWS_PW, CHUNK), jnp.int32),
            pltpu.VMEM((2 * GRP * CHUNK, dpad), jnp.float32),
            pltpu.SemaphoreType.DMA,
            pltpu.SemaphoreType.DMA,
        ],
        compiler_params=pltpu.CompilerParams(use_tc_tiling_on_sc=False),
        name=f"edge_scatter_d{dpad}",
    )
    return f(table, src2d, dst2d, zeros)


def _tc_matmul_body(x_ref, w_ref, o_ref):
    o_ref[...] = jnp.dot(x_ref[...], w_ref[...], preferred_element_type=jnp.float32)


def _tc_matmul(x, w):
    return pl.pallas_call(
        _tc_matmul_body,
        out_shape=jax.ShapeDtypeStruct((x.shape[0], w.shape[1]), jnp.float32),
    )(x, w)


def _tc_layer_body(h_ref, p_ref, ws_ref, wn_ref, b_ref, o_ref):
    msg = p_ref[0] + p_ref[1]
    acc = jnp.dot(h_ref[...], ws_ref[...], preferred_element_type=jnp.float32)
    acc = acc + jnp.dot(msg, wn_ref[...], preferred_element_type=jnp.float32)
    o_ref[...] = _selu(acc + b_ref[...])


def _tc_layer(h, p, ws, wn, b):
    return pl.pallas_call(
        _tc_layer_body,
        out_shape=jax.ShapeDtypeStruct((h.shape[0], wn.shape[1]), jnp.float32),
    )(h, p, ws, wn, b)


def _bn(v, gamma, beta):
    mu = jnp.mean(v, axis=0, keepdims=True)
    xc = v - mu
    var = jnp.mean(xc * xc, axis=0, keepdims=True)
    return gamma * xc / jnp.sqrt(var + 1e-5) + beta


def _tc_head_body(h3_ref, p_ref, ws_ref, wn_ref, b_ref,
                  mem_ref, wd_ref, bd_ref, w1_ref, b1_ref, ga1_ref, be1_ref,
                  w2_ref, b2_ref, ga2_ref, be2_ref, w3_ref, b3_ref, o_ref):
    msg = p_ref[0] + p_ref[1]
    h4 = jnp.dot(h3_ref[...], ws_ref[...], preferred_element_type=jnp.float32)
    h4 = h4 + jnp.dot(msg, wn_ref[...], preferred_element_type=jnp.float32)
    h4 = _selu(h4 + b_ref[...])
    h = h4[:N_NODES, :]
    mem = mem_ref[...]

    iog = lax.broadcasted_iota(jnp.int32, (N_NODES, N_GRAPHS), 1)
    onehot = (mem == iog).astype(jnp.float32)
    seg_sum = lax.dot_general(onehot, h, (((0,), (0,)), ((), ())),
                              preferred_element_type=jnp.float32)
    ones_col = jnp.ones((N_NODES, 1), jnp.float32)
    cnt = lax.dot_general(onehot, ones_col, (((0,), (0,)), ((), ())),
                          preferred_element_type=jnp.float32)

    neg = -1e30
    BLK = 16
    rib = lax.broadcasted_iota(jnp.int32, (N_NODES, 1), 0) % BLK
    fwd = h
    bwd = h
    for sh in (1, 2, 4, 8):
        s = sh
        m_up = jnp.concatenate([jnp.full((s, 1), -1, jnp.int32), mem[:-s]], axis=0)
        f_up = jnp.concatenate([jnp.full((s, h.shape[1]), neg, jnp.float32),
                                fwd[:-s]], axis=0)
        ok_f = (m_up == mem) & (rib >= s)
        fwd = jnp.maximum(fwd, jnp.where(ok_f, f_up, neg))
        m_dn = jnp.concatenate([mem[s:], jnp.full((s, 1), -1, jnp.int32)], axis=0)
        b_dn = jnp.concatenate([bwd[s:],
                                jnp.full((s, h.shape[1]), neg, jnp.float32)], axis=0)
        ok_b = (m_dn == mem) & (rib < BLK - s)
        bwd = jnp.maximum(bwd, jnp.where(ok_b, b_dn, neg))

    nblk = N_NODES // BLK
    blkmax = jnp.max(h.reshape(nblk, BLK, h.shape[1]), axis=1)
    mem_b = mem.reshape(nblk, BLK)
    bmin = jnp.min(mem_b, axis=1, keepdims=True)
    bmax = jnp.max(mem_b, axis=1, keepdims=True)

    first = jnp.concatenate([jnp.full((1, 1), -1, jnp.int32), mem[:-1]], axis=0) != mem
    last = jnp.concatenate([mem[1:], jnp.full((1, 1), -1, jnp.int32)], axis=0) != mem
    oh_first = (onehot * first.astype(jnp.float32))
    oh_last = (onehot * last.astype(jnp.float32))
    m_first = lax.dot_general(oh_first, bwd, (((0,), (0,)), ((), ())),
                              preferred_element_type=jnp.float32)
    m_last = lax.dot_general(oh_last, fwd, (((0,), (0,)), ((), ())),
                             preferred_element_type=jnp.float32)

    interior_rows = []
    for g in range(N_GRAPHS):
        sel = (bmin == g) & (bmax == g)
        vals = jnp.where(sel, blkmax, neg)
        interior_rows.append(jnp.max(vals, axis=0, keepdims=True))
    interior = jnp.concatenate(interior_rows, axis=0)

    seg_max = jnp.maximum(interior, jnp.maximum(m_first, m_last))
    seg_max = jnp.where(cnt > 0, seg_max, 0.0)

    ro = _selu(jnp.concatenate(
        [seg_sum[:, :N_FEAT_OUT], seg_max[:, :N_FEAT_OUT]], axis=1))
    logits = jnp.dot(ro, wd_ref[...], preferred_element_type=jnp.float32) + bd_ref[...]

    i24 = lax.broadcasted_iota(jnp.int32, (24, 24), 0)
    j24 = lax.broadcasted_iota(jnp.int32, (24, 24), 1)
    perm = ((i24 ^ 1) == j24).astype(jnp.float32)
    partner = jnp.dot(logits, perm, preferred_element_type=jnp.float32)
    mx = jnp.maximum(logits, partner)
    ea = jnp.exp(logits - mx)
    eb = jnp.exp(partner - mx)
    rd = ea / (ea + eb)

    f = jnp.dot(rd, w1_ref[...], preferred_element_type=jnp.float32) + b1_ref[...]
    f = jnp.maximum(_bn(f, ga1_ref[...], be1_ref[...]), 0.0)
    f = jnp.dot(f, w2_ref[...], preferred_element_type=jnp.float32) + b2_ref[...]
    f = jnp.maximum(_bn(f, ga2_ref[...], be2_ref[...]), 0.0)
    f = jnp.dot(f, w3_ref[...], preferred_element_type=jnp.float32) + b3_ref[...]
    o_ref[...] = 1.0 / (1.0 + jnp.exp(-f))


def _tc_head(h3, p4, ws4, wn4, b4p, memb, *ws):
    return pl.pallas_call(
        _tc_head_body,
        out_shape=jax.ShapeDtypeStruct((N_GRAPHS, ws[-2].shape[1]), jnp.float32),
        compiler_params=pltpu.CompilerParams(vmem_limit_bytes=60 * 1024 * 1024),
    )(h3, p4, ws4, wn4, b4p, memb, *ws)


def kernel(x, edge_index, membership,
           W_self1, W_nbr1, b1, W_self2, W_nbr2, b2,
           W_self3, W_nbr3, b3, W_self4, W_nbr4, b4,
           W_dense2, b_dense2,
           W1, b1_lin, gamma1, beta1,
           W2, b2_lin, gamma2, beta2,
           W3, b3_lin):
    ws1 = jnp.pad(W_self1, ((0, 0), (0, 1)))
    wn1 = jnp.pad(W_nbr1, ((0, 0), (0, 1)))
    b1p = jnp.pad(b1, (0, 1)).reshape(1, 16)
    ws2 = jnp.pad(W_self2, ((0, 1), (0, 12)))
    wn2 = jnp.pad(W_nbr2, ((0, 1), (0, 12)))
    b2p = jnp.pad(b2, (0, 12)).reshape(1, 32)
    ws3 = jnp.pad(W_self3, ((0, 12), (0, 5)))
    wn3 = jnp.pad(W_nbr3, ((0, 12), (0, 5)))
    b3p = jnp.pad(b3, (0, 5)).reshape(1, 32)
    ws4 = jnp.pad(W_self4, ((0, 5), (0, 28)))
    wn4 = jnp.pad(W_nbr4, ((0, 5), (0, 28)))
    b4p = jnp.pad(b4, (0, 28)).reshape(1, 64)
    eye16 = jnp.eye(16, dtype=jnp.float32)

    xp = jnp.pad(x, ((0, R - N_NODES), (0, 0)))
    pad_e = EROWS * CHUNK - N_EDGES
    pad_idx = jnp.full((pad_e,), N_NODES, jnp.int32)
    srcp = jnp.concatenate([edge_index[0], pad_idx]).reshape(EROWS, CHUNK)
    dstp = jnp.concatenate([edge_index[1], pad_idx]).reshape(EROWS, CHUNK)
    zeros16 = jnp.zeros((R, 16), jnp.float32)
    zeros32 = jnp.zeros((R, 32), jnp.float32)

    g1 = _tc_matmul(xp, wn1)
    p1 = _sc_scatter(16, g1, srcp, dstp, zeros16)
    h1 = _tc_layer(xp, p1, ws1, eye16, b1p)
    p2 = _sc_scatter(16, h1, srcp, dstp, zeros16)
    h2 = _tc_layer(h1, p2, ws2, wn2, b2p)
    p3 = _sc_scatter(32, h2, srcp, dstp, zeros32)
    h3 = _tc_layer(h2, p3, ws3, wn3, b3p)
    p4 = _sc_scatter(32, h3, srcp, dstp, zeros32)

    memb = membership.reshape(N_NODES, 1)
    return _tc_head(
        h3, p4, ws4, wn4, b4p, memb, W_dense2, b_dense2.reshape(1, -1),
        W1, b1_lin.reshape(1, -1), gamma1.reshape(1, -1), beta1.reshape(1, -1),
        W2, b2_lin.reshape(1, -1), gamma2.reshape(1, -1), beta2.reshape(1, -1),
        W3, b3_lin.reshape(1, -1))


# speedup vs baseline: 1.0662x; 1.0662x over previous
import functools

import jax
import jax.numpy as jnp
from jax import lax
from jax.experimental import pallas as pl
from jax.experimental.pallas import tpu as pltpu
from jax.experimental.pallas import tpu_sc as plsc

N_NODES = 10000
N_EDGES = 320000
N_GRAPHS = 64
N_FEAT_OUT = 36

R = 10112
NC, NS = 2, 16
NW = NC * NS
CHUNK = 128
EROWS = 2560
ROWS_PW = EROWS // NW
RPS = R // NS

_SELU_ALPHA = 1.6732632423543772
_SELU_SCALE = 1.0507009873554805


def _selu(v):
    return _SELU_SCALE * jnp.where(v > 0, v, _SELU_ALPHA * (jnp.exp(v) - 1.0))


GRP = 8


def _sc_scatter_body(dpad, table, src2d, dst2d, out, acc, sidx, didx, rows,
                     gsem, ssem):
    c = lax.axis_index("c")
    s = lax.axis_index("s")
    wid = s * NC + c

    @pl.loop(0, RPS)
    def _zrow(i):
        for jb in range(dpad // 16):
            rows[i, pl.ds(jb * 16, 16)] = jnp.zeros((16,), jnp.float32)

    pltpu.sync_copy(rows.at[pl.ds(0, RPS)], acc.at[pl.ds(s * RPS, RPS)])

    base = wid * ROWS_PW
    pltpu.sync_copy(src2d.at[pl.ds(base, ROWS_PW)], sidx)
    pltpu.sync_copy(dst2d.at[pl.ds(base, ROWS_PW)], didx)
    plsc.subcore_barrier()

    def _buf(half, j):
        return rows.at[pl.ds((half * GRP + j) * CHUNK, CHUNK)]

    def _fire_gathers(k0, half):
        return [
            pltpu.async_copy(table.at[sidx.at[k0 + j]], _buf(half, j), gsem)
            for j in range(GRP)
        ]

    def _scatter_half(k0, half, gd):
        for j in range(GRP):
            gd[j].wait()
            pltpu.async_copy(_buf(half, j), acc.at[didx.at[k0 + j]], ssem,
                             add=True)

    def _drain_scatters():
        for j in range(GRP):
            pltpu.make_async_copy(table.at[pl.ds(0, CHUNK)], _buf(0, j),
                                  ssem).wait()

    @pl.loop(0, ROWS_PW, step=2 * GRP)
    def _group(k0):
        @pl.when(k0 > 0)
        def _():
            _drain_scatters()
        gda = _fire_gathers(k0, 0)
        _scatter_half(k0, 0, gda)
        gdb = _fire_gathers(k0 + GRP, 1)
        _drain_scatters()
        _scatter_half(k0 + GRP, 1, gdb)

    _drain_scatters()

    plsc.subcore_barrier()
    pltpu.sync_copy(acc.at[pl.ds(s * RPS, RPS)], out.at[c, pl.ds(s * RPS, RPS)])


def _sc_scatter(dpad, table, src2d, dst2d):
    mesh = plsc.VectorSubcoreMesh(
        core_axis_name="c", subcore_axis_name="s", num_cores=NC, num_subcores=NS)
    f = pl.kernel(
        functools.partial(_sc_scatter_body, dpad),
        out_type=jax.ShapeDtypeStruct((NC, R, dpad), jnp.float32),
        mesh=mesh,
        scratch_types=[
            pltpu.VMEM_SHARED((R, dpad), jnp.float32),
            pltpu.VMEM((ROWS_PW, CHUNK), jnp.int32),
            pltpu.VMEM((ROWS_PW, CHUNK), jnp.int32),
            pltpu.VMEM((2 * GRP * CHUNK, dpad), jnp.float32),
            pltpu.SemaphoreType.DMA,
            pltpu.SemaphoreType.DMA,
        ],
        compiler_params=pltpu.CompilerParams(use_tc_tiling_on_sc=False),
        name=f"edge_scatter_d{dpad}",
    )
    return f(table, src2d, dst2d)


def _tc_matmul_body(x_ref, w_ref, o_ref):
    nh = x_ref.shape[0]
    o_ref[pl.ds(0, nh), :] = jnp.dot(x_ref[...], w_ref[...],
                                     preferred_element_type=jnp.float32)
    if nh < R:
        o_ref[pl.ds(nh, R - nh), :] = jnp.zeros((R - nh, o_ref.shape[1]),
                                                jnp.float32)


def _tc_matmul(x, w):
    return pl.pallas_call(
        _tc_matmul_body,
        out_shape=jax.ShapeDtypeStruct((R, w.shape[1]), jnp.float32),
    )(x, w)


def _tc_layer_body(h_ref, p_ref, ws_ref, wn_ref, b_ref, o_ref):
    nh = h_ref.shape[0]
    msg = p_ref[0][:nh] + p_ref[1][:nh]
    acc = jnp.dot(h_ref[...], ws_ref[...], preferred_element_type=jnp.float32)
    acc = acc + jnp.dot(msg, wn_ref[...], preferred_element_type=jnp.float32)
    o_ref[pl.ds(0, nh), :] = _selu(acc + b_ref[...])
    if nh < R:
        o_ref[pl.ds(nh, R - nh), :] = jnp.zeros((R - nh, o_ref.shape[1]),
                                                jnp.float32)


def _tc_layer(h, p, ws, wn, b):
    return pl.pallas_call(
        _tc_layer_body,
        out_shape=jax.ShapeDtypeStruct((R, wn.shape[1]), jnp.float32),
    )(h, p, ws, wn, b)


def _bn(v, gamma, beta):
    mu = jnp.mean(v, axis=0, keepdims=True)
    xc = v - mu
    var = jnp.mean(xc * xc, axis=0, keepdims=True)
    return gamma * xc / jnp.sqrt(var + 1e-5) + beta


def _tc_head_body(h3_ref, p_ref, ws_ref, wn_ref, b_ref,
                  mem_ref, wd_ref, bd_ref, w1_ref, b1_ref, ga1_ref, be1_ref,
                  w2_ref, b2_ref, ga2_ref, be2_ref, w3_ref, b3_ref, o_ref):
    msg = p_ref[0] + p_ref[1]
    h4 = jnp.dot(h3_ref[...], ws_ref[...], preferred_element_type=jnp.float32)
    h4 = h4 + jnp.dot(msg, wn_ref[...], preferred_element_type=jnp.float32)
    h4 = _selu(h4 + b_ref[...])
    h = h4[:N_NODES, :]
    mem = mem_ref[...]

    iog = lax.broadcasted_iota(jnp.int32, (N_NODES, N_GRAPHS), 1)
    onehot = (mem == iog).astype(jnp.float32)
    seg_sum = lax.dot_general(onehot, h, (((0,), (0,)), ((), ())),
                              preferred_element_type=jnp.float32)
    ones_col = jnp.ones((N_NODES, 1), jnp.float32)
    cnt = lax.dot_general(onehot, ones_col, (((0,), (0,)), ((), ())),
                          preferred_element_type=jnp.float32)

    neg = -1e30
    BLK = 16
    rib = lax.broadcasted_iota(jnp.int32, (N_NODES, 1), 0) % BLK
    fwd = h
    bwd = h
    for sh in (1, 2, 4, 8):
        s = sh
        m_up = jnp.concatenate([jnp.full((s, 1), -1, jnp.int32), mem[:-s]], axis=0)
        f_up = jnp.concatenate([jnp.full((s, h.shape[1]), neg, jnp.float32),
                                fwd[:-s]], axis=0)
        ok_f = (m_up == mem) & (rib >= s)
        fwd = jnp.maximum(fwd, jnp.where(ok_f, f_up, neg))
        m_dn = jnp.concatenate([mem[s:], jnp.full((s, 1), -1, jnp.int32)], axis=0)
        b_dn = jnp.concatenate([bwd[s:],
                                jnp.full((s, h.shape[1]), neg, jnp.float32)], axis=0)
        ok_b = (m_dn == mem) & (rib < BLK - s)
        bwd = jnp.maximum(bwd, jnp.where(ok_b, b_dn, neg))

    nblk = N_NODES // BLK
    blkmax = jnp.max(h.reshape(nblk, BLK, h.shape[1]), axis=1)
    mem_b = mem.reshape(nblk, BLK)
    bmin = jnp.min(mem_b, axis=1, keepdims=True)
    bmax = jnp.max(mem_b, axis=1, keepdims=True)

    first = jnp.concatenate([jnp.full((1, 1), -1, jnp.int32), mem[:-1]], axis=0) != mem
    last = jnp.concatenate([mem[1:], jnp.full((1, 1), -1, jnp.int32)], axis=0) != mem
    oh_first = (onehot * first.astype(jnp.float32))
    oh_last = (onehot * last.astype(jnp.float32))
    m_first = lax.dot_general(oh_first, bwd, (((0,), (0,)), ((), ())),
                              preferred_element_type=jnp.float32)
    m_last = lax.dot_general(oh_last, fwd, (((0,), (0,)), ((), ())),
                             preferred_element_type=jnp.float32)

    interior_rows = []
    for g in range(N_GRAPHS):
        sel = (bmin == g) & (bmax == g)
        vals = jnp.where(sel, blkmax, neg)
        interior_rows.append(jnp.max(vals, axis=0, keepdims=True))
    interior = jnp.concatenate(interior_rows, axis=0)

    seg_max = jnp.maximum(interior, jnp.maximum(m_first, m_last))
    seg_max = jnp.where(cnt > 0, seg_max, 0.0)

    ro = _selu(jnp.concatenate(
        [seg_sum[:, :N_FEAT_OUT], seg_max[:, :N_FEAT_OUT]], axis=1))
    logits = jnp.dot(ro, wd_ref[...], preferred_element_type=jnp.float32) + bd_ref[...]

    i24 = lax.broadcasted_iota(jnp.int32, (24, 24), 0)
    j24 = lax.broadcasted_iota(jnp.int32, (24, 24), 1)
    perm = ((i24 ^ 1) == j24).astype(jnp.float32)
    partner = jnp.dot(logits, perm, preferred_element_type=jnp.float32)
    mx = jnp.maximum(logits, partner)
    ea = jnp.exp(logits - mx)
    eb = jnp.exp(partner - mx)
    rd = ea / (ea + eb)

    f = jnp.dot(rd, w1_ref[...], preferred_element_type=jnp.float32) + b1_ref[...]
    f = jnp.maximum(_bn(f, ga1_ref[...], be1_ref[...]), 0.0)
    f = jnp.dot(f, w2_ref[...], preferred_element_type=jnp.float32) + b2_ref[...]
    f = jnp.maximum(_bn(f, ga2_ref[...], be2_ref[...]), 0.0)
    f = jnp.dot(f, w3_ref[...], preferred_element_type=jnp.float32) + b3_ref[...]
    o_ref[...] = 1.0 / (1.0 + jnp.exp(-f))


def _tc_head(h3, p4, ws4, wn4, b4p, memb, *ws):
    return pl.pallas_call(
        _tc_head_body,
        out_shape=jax.ShapeDtypeStruct((N_GRAPHS, ws[-2].shape[1]), jnp.float32),
        compiler_params=pltpu.CompilerParams(vmem_limit_bytes=60 * 1024 * 1024),
    )(h3, p4, ws4, wn4, b4p, memb, *ws)


def kernel(x, edge_index, membership,
           W_self1, W_nbr1, b1, W_self2, W_nbr2, b2,
           W_self3, W_nbr3, b3, W_self4, W_nbr4, b4,
           W_dense2, b_dense2,
           W1, b1_lin, gamma1, beta1,
           W2, b2_lin, gamma2, beta2,
           W3, b3_lin):
    ws1 = jnp.pad(W_self1, ((0, 0), (0, 1)))
    wn1 = jnp.pad(W_nbr1, ((0, 0), (0, 1)))
    b1p = jnp.pad(b1, (0, 1)).reshape(1, 16)
    ws2 = jnp.pad(W_self2, ((0, 1), (0, 12)))
    wn2 = jnp.pad(W_nbr2, ((0, 1), (0, 12)))
    b2p = jnp.pad(b2, (0, 12)).reshape(1, 32)
    ws3 = jnp.pad(W_self3, ((0, 12), (0, 5)))
    wn3 = jnp.pad(W_nbr3, ((0, 12), (0, 5)))
    b3p = jnp.pad(b3, (0, 5)).reshape(1, 32)
    ws4 = jnp.pad(W_self4, ((0, 5), (0, 28)))
    wn4 = jnp.pad(W_nbr4, ((0, 5), (0, 28)))
    b4p = jnp.pad(b4, (0, 28)).reshape(1, 64)
    eye16 = jnp.eye(16, dtype=jnp.float32)

    pad_e = EROWS * CHUNK - N_EDGES
    pad_idx = jnp.full((pad_e,), N_NODES, jnp.int32)
    srcp = jnp.concatenate([edge_index[0], pad_idx]).reshape(EROWS, CHUNK)
    dstp = jnp.concatenate([edge_index[1], pad_idx]).reshape(EROWS, CHUNK)

    g1 = _tc_matmul(x, wn1)
    p1 = _sc_scatter(16, g1, srcp, dstp)
    h1 = _tc_layer(x, p1, ws1, eye16, b1p)
    p2 = _sc_scatter(16, h1, srcp, dstp)
    h2 = _tc_layer(h1, p2, ws2, wn2, b2p)
    p3 = _sc_scatter(32, h2, srcp, dstp)
    h3 = _tc_layer(h2, p3, ws3, wn3, b3p)
    p4 = _sc_scatter(32, h3, srcp, dstp)

    memb = membership.reshape(N_NODES, 1)
    return _tc_head(
        h3, p4, ws4, wn4, b4p, memb, W_dense2, b_dense2.reshape(1, -1),
        W1, b1_lin.reshape(1, -1), gamma1.reshape(1, -1), beta1.reshape(1, -1),
        W2, b2_lin.reshape(1, -1), gamma2.reshape(1, -1), beta2.reshape(1, -1),
        W3, b3_lin.reshape(1, -1))


# speedup vs baseline: 1.8500x; 1.7351x over previous
import functools

import jax
import jax.numpy as jnp
from jax import lax
from jax.experimental import pallas as pl
from jax.experimental.pallas import tpu as pltpu
from jax.experimental.pallas import tpu_sc as plsc

N_NODES = 10000
N_EDGES = 320000
N_GRAPHS = 64
N_FEAT_OUT = 36

R = 10112
NC, NS = 2, 16
NW = NC * NS
CHUNK = 128
EROWS = 2560
ROWS_PW = EROWS // NW
RPS = R // NS

_SELU_ALPHA = 1.6732632423543772
_SELU_SCALE = 1.0507009873554805


def _selu(v):
    return _SELU_SCALE * jnp.where(v > 0, v, _SELU_ALPHA * (jnp.exp(v) - 1.0))


GRP = 8


def _sc_scatter_body(dpad, table, src2d, dst2d, out, acc, sidx, didx, rows,
                     gsem, ssem):
    c = lax.axis_index("c")
    s = lax.axis_index("s")
    wid = s * NC + c

    @pl.loop(0, RPS)
    def _zrow(i):
        for jb in range(dpad // 16):
            rows[i, pl.ds(jb * 16, 16)] = jnp.zeros((16,), jnp.float32)

    pltpu.sync_copy(rows.at[pl.ds(0, RPS)], acc.at[pl.ds(s * RPS, RPS)])

    base = wid * ROWS_PW
    pltpu.sync_copy(src2d.at[pl.ds(base, ROWS_PW)], sidx)
    pltpu.sync_copy(dst2d.at[pl.ds(base, ROWS_PW)], didx)
    plsc.subcore_barrier()

    def _buf(half, j):
        return rows.at[pl.ds((half * GRP + j) * CHUNK, CHUNK)]

    def _fire_gathers(k0, half):
        return [
            pltpu.async_copy(table.at[sidx.at[k0 + j]], _buf(half, j), gsem)
            for j in range(GRP)
        ]

    def _scatter_half(k0, half, gd):
        for j in range(GRP):
            gd[j].wait()
            pltpu.async_copy(_buf(half, j), acc.at[didx.at[k0 + j]], ssem,
                             add=True)

    def _drain_scatters():
        for j in range(GRP):
            pltpu.make_async_copy(table.at[pl.ds(0, CHUNK)], _buf(0, j),
                                  ssem).wait()

    @pl.loop(0, ROWS_PW, step=2 * GRP)
    def _group(k0):
        @pl.when(k0 > 0)
        def _():
            _drain_scatters()
        gda = _fire_gathers(k0, 0)
        _scatter_half(k0, 0, gda)
        gdb = _fire_gathers(k0 + GRP, 1)
        _drain_scatters()
        _scatter_half(k0 + GRP, 1, gdb)

    _drain_scatters()

    plsc.subcore_barrier()
    pltpu.sync_copy(acc.at[pl.ds(s * RPS, RPS)], out.at[c, pl.ds(s * RPS, RPS)])


def _sc_scatter(dpad, table, src2d, dst2d):
    mesh = plsc.VectorSubcoreMesh(
        core_axis_name="c", subcore_axis_name="s", num_cores=NC, num_subcores=NS)
    f = pl.kernel(
        functools.partial(_sc_scatter_body, dpad),
        out_type=jax.ShapeDtypeStruct((NC, R, dpad), jnp.float32),
        mesh=mesh,
        scratch_types=[
            pltpu.VMEM_SHARED((R, dpad), jnp.float32),
            pltpu.VMEM((ROWS_PW, CHUNK), jnp.int32),
            pltpu.VMEM((ROWS_PW, CHUNK), jnp.int32),
            pltpu.VMEM((2 * GRP * CHUNK, dpad), jnp.float32),
            pltpu.SemaphoreType.DMA,
            pltpu.SemaphoreType.DMA,
        ],
        compiler_params=pltpu.CompilerParams(use_tc_tiling_on_sc=False),
        name=f"edge_scatter_d{dpad}",
    )
    return f(table, src2d, dst2d)


def _tc_matmul_body(x_ref, w_ref, o_ref):
    nh = x_ref.shape[0]
    o_ref[pl.ds(0, nh), :] = jnp.dot(x_ref[...], w_ref[...],
                                     preferred_element_type=jnp.float32)
    if nh < R:
        o_ref[pl.ds(nh, R - nh), :] = jnp.zeros((R - nh, o_ref.shape[1]),
                                                jnp.float32)


def _tc_matmul(x, w):
    return pl.pallas_call(
        _tc_matmul_body,
        out_shape=jax.ShapeDtypeStruct((R, w.shape[1]), jnp.float32),
    )(x, w)


def _tc_layer_body(h_ref, p_ref, ws_ref, wn_ref, b_ref, o_ref):
    nh = h_ref.shape[0]
    msg = p_ref[0][:nh] + p_ref[1][:nh]
    acc = jnp.dot(h_ref[...], ws_ref[...], preferred_element_type=jnp.float32)
    acc = acc + jnp.dot(msg, wn_ref[...], preferred_element_type=jnp.float32)
    o_ref[pl.ds(0, nh), :] = _selu(acc + b_ref[...])
    if nh < R:
        o_ref[pl.ds(nh, R - nh), :] = jnp.zeros((R - nh, o_ref.shape[1]),
                                                jnp.float32)


def _tc_layer(h, p, ws, wn, b):
    return pl.pallas_call(
        _tc_layer_body,
        out_shape=jax.ShapeDtypeStruct((R, wn.shape[1]), jnp.float32),
    )(h, p, ws, wn, b)


def _bn(v, gamma, beta):
    mu = jnp.mean(v, axis=0, keepdims=True)
    xc = v - mu
    var = jnp.mean(xc * xc, axis=0, keepdims=True)
    return gamma * xc / jnp.sqrt(var + 1e-5) + beta


def _tc_head_body(h3_ref, p_ref, ws_ref, wn_ref, b_ref,
                  mem_ref, wd_ref, bd_ref, w1_ref, b1_ref, ga1_ref, be1_ref,
                  w2_ref, b2_ref, ga2_ref, be2_ref, w3_ref, b3_ref, o_ref):
    msg = p_ref[0] + p_ref[1]
    h4 = jnp.dot(h3_ref[...], ws_ref[...], preferred_element_type=jnp.float32)
    h4 = h4 + jnp.dot(msg, wn_ref[...], preferred_element_type=jnp.float32)
    h4 = _selu(h4 + b_ref[...])
    h = h4[:N_NODES, :]
    mem = mem_ref[...]

    iog = lax.broadcasted_iota(jnp.int32, (N_NODES, N_GRAPHS), 1)
    onehot = (mem == iog).astype(jnp.float32)
    seg_sum = lax.dot_general(onehot, h, (((0,), (0,)), ((), ())),
                              preferred_element_type=jnp.float32)
    ones_col = jnp.ones((N_NODES, 1), jnp.float32)
    cnt = lax.dot_general(onehot, ones_col, (((0,), (0,)), ((), ())),
                          preferred_element_type=jnp.float32)

    neg = -1e30
    BLK = 16
    rib = lax.broadcasted_iota(jnp.int32, (N_NODES, 1), 0) % BLK
    fwd = h
    bwd = h
    for sh in (1, 2, 4, 8):
        s = sh
        m_up = jnp.concatenate([jnp.full((s, 1), -1, jnp.int32), mem[:-s]], axis=0)
        f_up = jnp.concatenate([jnp.full((s, h.shape[1]), neg, jnp.float32),
                                fwd[:-s]], axis=0)
        ok_f = (m_up == mem) & (rib >= s)
        fwd = jnp.maximum(fwd, jnp.where(ok_f, f_up, neg))
        m_dn = jnp.concatenate([mem[s:], jnp.full((s, 1), -1, jnp.int32)], axis=0)
        b_dn = jnp.concatenate([bwd[s:],
                                jnp.full((s, h.shape[1]), neg, jnp.float32)], axis=0)
        ok_b = (m_dn == mem) & (rib < BLK - s)
        bwd = jnp.maximum(bwd, jnp.where(ok_b, b_dn, neg))

    nblk = N_NODES // BLK
    blkmax = jnp.max(h.reshape(nblk, BLK, h.shape[1]), axis=1)
    mem_b = mem.reshape(nblk, BLK)
    bmin = jnp.min(mem_b, axis=1, keepdims=True)
    bmax = jnp.max(mem_b, axis=1, keepdims=True)

    first = jnp.concatenate([jnp.full((1, 1), -1, jnp.int32), mem[:-1]], axis=0) != mem
    last = jnp.concatenate([mem[1:], jnp.full((1, 1), -1, jnp.int32)], axis=0) != mem
    oh_first = (onehot * first.astype(jnp.float32))
    oh_last = (onehot * last.astype(jnp.float32))
    m_first = lax.dot_general(oh_first, bwd, (((0,), (0,)), ((), ())),
                              preferred_element_type=jnp.float32)
    m_last = lax.dot_general(oh_last, fwd, (((0,), (0,)), ((), ())),
                             preferred_element_type=jnp.float32)

    interior_rows = []
    for g in range(N_GRAPHS):
        sel = (bmin == g) & (bmax == g)
        vals = jnp.where(sel, blkmax, neg)
        interior_rows.append(jnp.max(vals, axis=0, keepdims=True))
    interior = jnp.concatenate(interior_rows, axis=0)

    seg_max = jnp.maximum(interior, jnp.maximum(m_first, m_last))
    seg_max = jnp.where(cnt > 0, seg_max, 0.0)

    ro = _selu(jnp.concatenate(
        [seg_sum[:, :N_FEAT_OUT], seg_max[:, :N_FEAT_OUT]], axis=1))
    logits = jnp.dot(ro, wd_ref[...], preferred_element_type=jnp.float32) + bd_ref[...]

    i24 = lax.broadcasted_iota(jnp.int32, (24, 24), 0)
    j24 = lax.broadcasted_iota(jnp.int32, (24, 24), 1)
    perm = ((i24 ^ 1) == j24).astype(jnp.float32)
    partner = jnp.dot(logits, perm, preferred_element_type=jnp.float32)
    mx = jnp.maximum(logits, partner)
    ea = jnp.exp(logits - mx)
    eb = jnp.exp(partner - mx)
    rd = ea / (ea + eb)

    f = jnp.dot(rd, w1_ref[...], preferred_element_type=jnp.float32) + b1_ref[...]
    f = jnp.maximum(_bn(f, ga1_ref[...], be1_ref[...]), 0.0)
    f = jnp.dot(f, w2_ref[...], preferred_element_type=jnp.float32) + b2_ref[...]
    f = jnp.maximum(_bn(f, ga2_ref[...], be2_ref[...]), 0.0)
    f = jnp.dot(f, w3_ref[...], preferred_element_type=jnp.float32) + b3_ref[...]
    o_ref[...] = 1.0 / (1.0 + jnp.exp(-f))


def _tc_head(h3, p4, ws4, wn4, b4p, memb, *ws):
    return pl.pallas_call(
        _tc_head_body,
        out_shape=jax.ShapeDtypeStruct((N_GRAPHS, ws[-2].shape[1]), jnp.float32),
        compiler_params=pltpu.CompilerParams(vmem_limit_bytes=60 * 1024 * 1024),
    )(h3, p4, ws4, wn4, b4p, memb, *ws)


def kernel(x, edge_index, membership,
           W_self1, W_nbr1, b1, W_self2, W_nbr2, b2,
           W_self3, W_nbr3, b3, W_self4, W_nbr4, b4,
           W_dense2, b_dense2,
           W1, b1_lin, gamma1, beta1,
           W2, b2_lin, gamma2, beta2,
           W3, b3_lin):
    ws1 = jnp.pad(W_self1, ((0, 0), (0, 1)))
    wn1 = jnp.pad(W_nbr1, ((0, 0), (0, 1)))
    b1p = jnp.pad(b1, (0, 1)).reshape(1, 16)
    ws2 = jnp.pad(W_self2, ((0, 1), (0, 12)))
    wn2 = jnp.pad(W_nbr2, ((0, 1), (0, 12)))
    b2p = jnp.pad(b2, (0, 12)).reshape(1, 32)
    ws3 = jnp.pad(W_self3, ((0, 12), (0, 5)))
    wn3 = jnp.pad(W_nbr3, ((0, 12), (0, 5)))
    b3p = jnp.pad(b3, (0, 5)).reshape(1, 32)
    ws4 = jnp.pad(W_self4, ((0, 5), (0, 28)))
    wn4 = jnp.pad(W_nbr4, ((0, 5), (0, 28)))
    b4p = jnp.pad(b4, (0, 28)).reshape(1, 64)
    eye16 = jnp.eye(16, dtype=jnp.float32)

    pad_e = EROWS * CHUNK - N_EDGES
    pad_idx = N_NODES + (jnp.arange(pad_e, dtype=jnp.int32) % (R - N_NODES))
    srcp = jnp.concatenate([edge_index[0], pad_idx]).reshape(EROWS, CHUNK)
    dstp = jnp.concatenate([edge_index[1], pad_idx]).reshape(EROWS, CHUNK)

    g1 = _tc_matmul(x, wn1)
    p1 = _sc_scatter(16, g1, srcp, dstp)
    h1 = _tc_layer(x, p1, ws1, eye16, b1p)
    p2 = _sc_scatter(16, h1, srcp, dstp)
    h2 = _tc_layer(h1, p2, ws2, wn2, b2p)
    p3 = _sc_scatter(32, h2, srcp, dstp)
    h3 = _tc_layer(h2, p3, ws3, wn3, b3p)
    p4 = _sc_scatter(32, h3, srcp, dstp)

    memb = membership.reshape(N_NODES, 1)
    return _tc_head(
        h3, p4, ws4, wn4, b4p, memb, W_dense2, b_dense2.reshape(1, -1),
        W1, b1_lin.reshape(1, -1), gamma1.reshape(1, -1), beta1.reshape(1, -1),
        W2, b2_lin.reshape(1, -1), gamma2.reshape(1, -1), beta2.reshape(1, -1),
        W3, b3_lin.reshape(1, -1))
